# baseline (speedup 1.0000x reference)
import functools

import jax
import jax.numpy as jnp
from jax import lax
from jax.experimental import pallas as pl
from jax.experimental.pallas import tpu as pltpu
from jax.experimental.pallas import tpu_sc as plsc

N = 10000
E = 100000
D = 32
DE = 16
DH = 128
STEPS = 2

NC = 2
NS = 16
NW = NC * NS
GCH = 128
NCH = 25
EPW = NCH * GCH
E_PAD = NW * EPW
N_ACC = 10112
RPS = N_ACC // NS
TE = 1024


@functools.lru_cache(maxsize=1)
def _sc_kernels():
    mesh = plsc.VectorSubcoreMesh(core_axis_name="c", subcore_axis_name="s")

    @functools.partial(
        pl.kernel,
        mesh=mesh,
        out_type=jax.ShapeDtypeStruct((E_PAD, D), jnp.float32),
        scratch_types=[
            pltpu.VMEM((NCH, GCH), jnp.int32),
            pltpu.VMEM((EPW, D), jnp.float32),
            pltpu.SemaphoreType.DMA,
        ],
        compiler_params=pltpu.CompilerParams(use_tc_tiling_on_sc=False),
    )
    def _sc_gather(nodes_hbm, src_hbm, hsrc_hbm, idx_v, rows_v, sem):
        c = lax.axis_index("c")
        s = lax.axis_index("s")
        wid = s * NC + c
        base = wid * EPW
        pltpu.sync_copy(src_hbm.at[wid], idx_v)

        def _issue(j, carry):
            pltpu.async_copy(nodes_hbm.at[idx_v.at[j]],
                             rows_v.at[pl.ds(j * GCH, GCH)], sem)
            return carry

        lax.fori_loop(0, NCH, _issue, 0)

        def _drain(j, carry):
            pltpu.make_async_copy(nodes_hbm.at[idx_v.at[0]],
                                  rows_v.at[pl.ds(0, GCH)], sem).wait()
            return carry

        lax.fori_loop(0, NCH, _drain, 0)
        pltpu.sync_copy(rows_v, hsrc_hbm.at[pl.ds(base, EPW)])

    @functools.partial(
        pl.kernel,
        mesh=mesh,
        out_type=jax.ShapeDtypeStruct((NC, N_ACC, D), jnp.float32),
        scratch_types=[
            pltpu.VMEM((NCH, GCH), jnp.int32),
            pltpu.VMEM((EPW, D), jnp.float32),
            pltpu.VMEM_SHARED((N_ACC, D), jnp.float32),
            pltpu.SemaphoreType.DMA,
        ],
        compiler_params=pltpu.CompilerParams(use_tc_tiling_on_sc=False),
    )
    def _sc_scatter(msg_hbm, dst_hbm, zeros_hbm, part_hbm,
                    idx_v, rows_v, acc_sh, sem):
        c = lax.axis_index("c")
        s = lax.axis_index("s")
        wid = c * NS + s
        base = wid * EPW

        pltpu.sync_copy(zeros_hbm.at[pl.ds(s * RPS, RPS)],
                        acc_sh.at[pl.ds(s * RPS, RPS)])
        plsc.subcore_barrier()

        pltpu.sync_copy(dst_hbm.at[wid], idx_v)
        pltpu.sync_copy(msg_hbm.at[pl.ds(base, EPW)], rows_v)

        def _scat(j, carry):
            pltpu.sync_copy(rows_v.at[pl.ds(j * GCH, GCH)],
                            acc_sh.at[idx_v.at[j]], add=True)
            return carry

        lax.fori_loop(0, NCH, _scat, 0)
        plsc.subcore_barrier()

        pltpu.sync_copy(acc_sh.at[pl.ds(s * RPS, RPS)],
                        part_hbm.at[c, pl.ds(s * RPS, RPS)])

    return _sc_gather, _sc_scatter


def _msg_body(ef_ref, hs_ref, we1_ref, be1_ref, we2_ref, be2_ref, msg_ref):
    henc = jnp.maximum(
        jnp.dot(ef_ref[...], we1_ref[...], preferred_element_type=jnp.float32)
        + be1_ref[...], 0.0)
    ewt = jnp.dot(henc, we2_ref[...],
                  preferred_element_type=jnp.float32) + be2_ref[...]
    acc = hs_ref[:, 0:1] * ewt[:, 0:D]
    for i in range(1, D):
        acc = acc + hs_ref[:, i:i + 1] * ewt[:, i * D:(i + 1) * D]
    msg_ref[...] = acc


def _msg_kernel(e_feat_p, h_src, We1, be1, We2, be2):
    grid = (E_PAD // TE,)
    return pl.pallas_call(
        _msg_body,
        grid=grid,
        in_specs=[
            pl.BlockSpec((TE, DE), lambda i: (i, 0)),
            pl.BlockSpec((TE, D), lambda i: (i, 0)),
            pl.BlockSpec((DE, DH), lambda i: (0, 0)),
            pl.BlockSpec((1, DH), lambda i: (0, 0)),
            pl.BlockSpec((DH, D * D), lambda i: (0, 0)),
            pl.BlockSpec((1, D * D), lambda i: (0, 0)),
        ],
        out_specs=pl.BlockSpec((TE, D), lambda i: (i, 0)),
        out_shape=jax.ShapeDtypeStruct((E_PAD, D), jnp.float32),
    )(e_feat_p, h_src, We1, be1, We2, be2)


def _prologue_body(nf_ref, w0_ref, b0_ref, out_ref):
    out_ref[...] = jnp.maximum(
        jnp.dot(nf_ref[...], w0_ref[...], preferred_element_type=jnp.float32)
        + b0_ref[...], 0.0)


def _prologue(n_feat, W0, b0):
    return pl.pallas_call(
        _prologue_body,
        out_shape=jax.ShapeDtypeStruct((N, D), jnp.float32),
    )(n_feat, W0, b0)


def _update_body(p0_ref, p1_ref, out_ref, cb_ref, wm1_ref, wm2_ref, bm_ref,
                 o_ref):
    out = out_ref[...]
    m = jnp.maximum(p0_ref[...] + p1_ref[...] + out + cb_ref[...], 0.0)
    o_ref[...] = (jnp.dot(m, wm1_ref[...], preferred_element_type=jnp.float32)
                  + jnp.dot(out, wm2_ref[...],
                            preferred_element_type=jnp.float32)
                  + bm_ref[...])


def _final_body(p0_ref, p1_ref, out_ref, cb_ref, wm1_ref, wm2_ref, bm_ref,
                init_ref, o_ref):
    out = out_ref[...]
    m = jnp.maximum(p0_ref[...] + p1_ref[...] + out + cb_ref[...], 0.0)
    o_ref[...] = (jnp.dot(m, wm1_ref[...], preferred_element_type=jnp.float32)
                  + jnp.dot(out, wm2_ref[...],
                            preferred_element_type=jnp.float32)
                  + bm_ref[...] + init_ref[...])


def _update(p0, p1, out, cb, wm1, wm2, bm):
    return pl.pallas_call(
        _update_body,
        out_shape=jax.ShapeDtypeStruct((N, D), jnp.float32),
    )(p0, p1, out, cb, wm1, wm2, bm)


def _final(p0, p1, out, cb, wm1, wm2, bm, init):
    return pl.pallas_call(
        _final_body,
        out_shape=jax.ShapeDtypeStruct((N, D), jnp.float32),
    )(p0, p1, out, cb, wm1, wm2, bm, init)


def kernel(edge_index, n_feat, e_feat, W0, b0, We1, be1, We2, be2, conv_bias,
           Wm, bm):
    src = edge_index[0]
    dst = edge_index[1]
    npad = E_PAD - E
    fill = (jnp.arange(npad, dtype=jnp.int32) % (N_ACC - N))
    src_p = jnp.concatenate([src, fill]).reshape(NW, NCH, GCH)
    dst_p = jnp.concatenate([dst, N + fill]).reshape(NW, NCH, GCH)
    e_feat_p = jnp.concatenate(
        [e_feat, jnp.zeros((npad, DE), jnp.float32)], axis=0)
    zeros_acc = jnp.zeros((N_ACC, D), jnp.float32)
    be1_2 = be1.reshape(1, DH)
    be2_2 = be2.reshape(1, D * D)
    cb_2 = conv_bias.reshape(1, D)
    bm_2 = bm.reshape(1, D)
    b0_2 = b0.reshape(1, D)
    wm1 = Wm[:D]
    wm2 = Wm[D:]

    sc_gather, sc_scatter = _sc_kernels()
    out = _prologue(n_feat, W0, b0_2)
    for step in range(STEPS):
        h_src = sc_gather(out, src_p)
        msg = _msg_kernel(e_feat_p, h_src, We1, be1_2, We2, be2_2)
        parts = sc_scatter(msg, dst_p, zeros_acc)
        p0 = parts[0, :N]
        p1 = parts[1, :N]
        if step == STEPS - 1:
            out = _final(p0, p1, out, cb_2, wm1, wm2, bm_2, n_feat)
        else:
            out = _update(p0, p1, out, cb_2, wm1, wm2, bm_2)
    return out


# speedup vs baseline: 2.5708x; 2.5708x over previous
import functools

import jax
import jax.numpy as jnp
from jax import lax
from jax.experimental import pallas as pl
from jax.experimental.pallas import tpu as pltpu
from jax.experimental.pallas import tpu_sc as plsc

N = 10000
E = 100000
D = 32
DE = 16
DH = 128
STEPS = 2

NC = 2
NS = 16
NW = NC * NS
GCH = 128
NCH = 25
EPW = NCH * GCH
E_PAD = NW * EPW
N_ACC = 10112
RPS = N_ACC // NS
TE = 1024


@functools.lru_cache(maxsize=1)
def _sc_kernels():
    mesh = plsc.VectorSubcoreMesh(core_axis_name="c", subcore_axis_name="s")

    @functools.partial(
        pl.kernel,
        mesh=mesh,
        out_type=jax.ShapeDtypeStruct((E_PAD, D), jnp.float32),
        scratch_types=[
            pltpu.VMEM((NCH, GCH), jnp.int32),
            pltpu.VMEM((EPW, D), jnp.float32),
            pltpu.SemaphoreType.DMA,
        ],
        compiler_params=pltpu.CompilerParams(use_tc_tiling_on_sc=False),
    )
    def _sc_gather(nodes_hbm, src_hbm, hsrc_hbm, idx_v, rows_v, sem):
        c = lax.axis_index("c")
        s = lax.axis_index("s")
        wid = s * NC + c
        base = wid * EPW
        pltpu.sync_copy(src_hbm.at[wid], idx_v)

        def _issue(j, carry):
            pltpu.async_copy(nodes_hbm.at[idx_v.at[j]],
                             rows_v.at[pl.ds(j * GCH, GCH)], sem)
            return carry

        lax.fori_loop(0, NCH, _issue, 0)

        def _drain(j, carry):
            pltpu.make_async_copy(nodes_hbm.at[idx_v.at[0]],
                                  rows_v.at[pl.ds(0, GCH)], sem).wait()
            return carry

        lax.fori_loop(0, NCH, _drain, 0)
        pltpu.sync_copy(rows_v, hsrc_hbm.at[pl.ds(base, EPW)])

    @functools.partial(
        pl.kernel,
        mesh=mesh,
        out_type=jax.ShapeDtypeStruct((NC, N_ACC, D), jnp.float32),
        scratch_types=[
            pltpu.VMEM((NCH, GCH), jnp.int32),
            pltpu.VMEM((EPW, D), jnp.float32),
            pltpu.VMEM_SHARED((N_ACC, D), jnp.float32),
            pltpu.SemaphoreType.DMA,
        ],
        compiler_params=pltpu.CompilerParams(use_tc_tiling_on_sc=False),
    )
    def _sc_scatter(msg_hbm, dst_hbm, zeros_hbm, part_hbm,
                    idx_v, rows_v, acc_sh, sem):
        c = lax.axis_index("c")
        s = lax.axis_index("s")
        wid = c * NS + s
        base = wid * EPW

        pltpu.sync_copy(zeros_hbm.at[pl.ds(s * RPS, RPS)],
                        acc_sh.at[pl.ds(s * RPS, RPS)])
        plsc.subcore_barrier()

        pltpu.sync_copy(dst_hbm.at[wid], idx_v)
        pltpu.sync_copy(msg_hbm.at[pl.ds(base, EPW)], rows_v)

        def _scat(j, carry):
            pltpu.sync_copy(rows_v.at[pl.ds(j * GCH, GCH)],
                            acc_sh.at[idx_v.at[j]], add=True)
            return carry

        lax.fori_loop(0, NCH, _scat, 0)
        plsc.subcore_barrier()

        pltpu.sync_copy(acc_sh.at[pl.ds(s * RPS, RPS)],
                        part_hbm.at[c, pl.ds(s * RPS, RPS)])

    return _sc_gather, _sc_scatter


def _msg_body(ef_ref, hs_ref, we1_ref, be1_ref, we2_ref, be2_ref, p_ref,
              s_ref, msg_ref):
    henc = jnp.maximum(
        jnp.dot(ef_ref[...], we1_ref[...], preferred_element_type=jnp.float32)
        + be1_ref[...], 0.0)
    ewt = jnp.dot(henc, we2_ref[...],
                  preferred_element_type=jnp.float32) + be2_ref[...]
    hrep = jnp.dot(hs_ref[...], p_ref[...], preferred_element_type=jnp.float32)
    msg_ref[...] = jnp.dot(hrep * ewt, s_ref[...],
                           preferred_element_type=jnp.float32)


def _msg_kernel(e_feat_p, h_src, We1, be1, We2, be2, P, S):
    grid = (E_PAD // TE,)
    return pl.pallas_call(
        _msg_body,
        grid=grid,
        in_specs=[
            pl.BlockSpec((TE, DE), lambda i: (i, 0)),
            pl.BlockSpec((TE, D), lambda i: (i, 0)),
            pl.BlockSpec((DE, DH), lambda i: (0, 0)),
            pl.BlockSpec((1, DH), lambda i: (0, 0)),
            pl.BlockSpec((DH, D * D), lambda i: (0, 0)),
            pl.BlockSpec((1, D * D), lambda i: (0, 0)),
            pl.BlockSpec((D, D * D), lambda i: (0, 0)),
            pl.BlockSpec((D * D, D), lambda i: (0, 0)),
        ],
        out_specs=pl.BlockSpec((TE, D), lambda i: (i, 0)),
        out_shape=jax.ShapeDtypeStruct((E_PAD, D), jnp.float32),
    )(e_feat_p, h_src, We1, be1, We2, be2, P, S)


def _prologue_body(nf_ref, w0_ref, b0_ref, out_ref):
    out_ref[...] = jnp.maximum(
        jnp.dot(nf_ref[...], w0_ref[...], preferred_element_type=jnp.float32)
        + b0_ref[...], 0.0)


def _prologue(n_feat, W0, b0):
    return pl.pallas_call(
        _prologue_body,
        out_shape=jax.ShapeDtypeStruct((N, D), jnp.float32),
    )(n_feat, W0, b0)


def _update_body(p0_ref, p1_ref, out_ref, cb_ref, wm1_ref, wm2_ref, bm_ref,
                 o_ref):
    out = out_ref[...]
    m = jnp.maximum(p0_ref[...] + p1_ref[...] + out + cb_ref[...], 0.0)
    o_ref[...] = (jnp.dot(m, wm1_ref[...], preferred_element_type=jnp.float32)
                  + jnp.dot(out, wm2_ref[...],
                            preferred_element_type=jnp.float32)
                  + bm_ref[...])


def _final_body(p0_ref, p1_ref, out_ref, cb_ref, wm1_ref, wm2_ref, bm_ref,
                init_ref, o_ref):
    out = out_ref[...]
    m = jnp.maximum(p0_ref[...] + p1_ref[...] + out + cb_ref[...], 0.0)
    o_ref[...] = (jnp.dot(m, wm1_ref[...], preferred_element_type=jnp.float32)
                  + jnp.dot(out, wm2_ref[...],
                            preferred_element_type=jnp.float32)
                  + bm_ref[...] + init_ref[...])


def _update(p0, p1, out, cb, wm1, wm2, bm):
    return pl.pallas_call(
        _update_body,
        out_shape=jax.ShapeDtypeStruct((N, D), jnp.float32),
    )(p0, p1, out, cb, wm1, wm2, bm)


def _final(p0, p1, out, cb, wm1, wm2, bm, init):
    return pl.pallas_call(
        _final_body,
        out_shape=jax.ShapeDtypeStruct((N, D), jnp.float32),
    )(p0, p1, out, cb, wm1, wm2, bm, init)


def kernel(edge_index, n_feat, e_feat, W0, b0, We1, be1, We2, be2, conv_bias,
           Wm, bm):
    src = edge_index[0]
    dst = edge_index[1]
    npad = E_PAD - E
    fill = (jnp.arange(npad, dtype=jnp.int32) % (N_ACC - N))
    src_p = jnp.concatenate([src, fill]).reshape(NW, NCH, GCH)
    dst_p = jnp.concatenate([dst, N + fill]).reshape(NW, NCH, GCH)
    e_feat_p = jnp.concatenate(
        [e_feat, jnp.zeros((npad, DE), jnp.float32)], axis=0)
    zeros_acc = jnp.zeros((N_ACC, D), jnp.float32)
    be1_2 = be1.reshape(1, DH)
    be2_2 = be2.reshape(1, D * D)
    cb_2 = conv_bias.reshape(1, D)
    bm_2 = bm.reshape(1, D)
    b0_2 = b0.reshape(1, D)
    wm1 = Wm[:D]
    wm2 = Wm[D:]
    eye = jnp.eye(D, dtype=jnp.float32)
    P = jnp.kron(eye, jnp.ones((1, D), jnp.float32))
    S = jnp.kron(jnp.ones((D, 1), jnp.float32), eye)

    sc_gather, sc_scatter = _sc_kernels()
    out = _prologue(n_feat, W0, b0_2)
    for step in range(STEPS):
        h_src = sc_gather(out, src_p)
        msg = _msg_kernel(e_feat_p, h_src, We1, be1_2, We2, be2_2, P, S)
        parts = sc_scatter(msg, dst_p, zeros_acc)
        p0 = parts[0, :N]
        p1 = parts[1, :N]
        if step == STEPS - 1:
            out = _final(p0, p1, out, cb_2, wm1, wm2, bm_2, n_feat)
        else:
            out = _update(p0, p1, out, cb_2, wm1, wm2, bm_2)
    return out


# speedup vs baseline: 3.1255x; 1.2158x over previous
import functools

import jax
import jax.numpy as jnp
from jax import lax
from jax.experimental import pallas as pl
from jax.experimental.pallas import tpu as pltpu
from jax.experimental.pallas import tpu_sc as plsc

N = 10000
E = 100000
D = 32
DE = 16
DH = 128
STEPS = 2

NC = 2
NS = 16
NW = NC * NS
GCH = 128
NCH = 25
EPW = NCH * GCH
E_PAD = NW * EPW
N_ACC = 10112
RPS = N_ACC // NS
TE = 1024


@functools.lru_cache(maxsize=1)
def _sc_kernels():
    mesh = plsc.VectorSubcoreMesh(core_axis_name="c", subcore_axis_name="s")

    @functools.partial(
        pl.kernel,
        mesh=mesh,
        out_type=jax.ShapeDtypeStruct((E_PAD, D), jnp.float32),
        scratch_types=[
            pltpu.VMEM((NCH, GCH), jnp.int32),
            pltpu.VMEM((EPW, D), jnp.float32),
            pltpu.SemaphoreType.DMA,
        ],
        compiler_params=pltpu.CompilerParams(use_tc_tiling_on_sc=False),
    )
    def _sc_gather(nodes_hbm, src_hbm, hsrc_hbm, idx_v, rows_v, sem):
        c = lax.axis_index("c")
        s = lax.axis_index("s")
        wid = s * NC + c
        base = wid * EPW
        pltpu.sync_copy(src_hbm.at[wid], idx_v)

        def _issue(j, carry):
            pltpu.async_copy(nodes_hbm.at[idx_v.at[j]],
                             rows_v.at[pl.ds(j * GCH, GCH)], sem)
            return carry

        lax.fori_loop(0, NCH, _issue, 0)

        def _drain(j, carry):
            pltpu.make_async_copy(nodes_hbm.at[idx_v.at[0]],
                                  rows_v.at[pl.ds(0, GCH)], sem).wait()
            return carry

        lax.fori_loop(0, NCH, _drain, 0)
        pltpu.sync_copy(rows_v, hsrc_hbm.at[pl.ds(base, EPW)])

    @functools.partial(
        pl.kernel,
        mesh=mesh,
        out_type=jax.ShapeDtypeStruct((NC, N_ACC, D), jnp.float32),
        scratch_types=[
            pltpu.VMEM((NCH, GCH), jnp.int32),
            pltpu.VMEM((EPW, D), jnp.float32),
            pltpu.VMEM_SHARED((N_ACC, D), jnp.float32),
            pltpu.SemaphoreType.DMA,
        ],
        compiler_params=pltpu.CompilerParams(use_tc_tiling_on_sc=False),
    )
    def _sc_scatter(msg_hbm, dst_hbm, zeros_hbm, part_hbm,
                    idx_v, rows_v, acc_sh, sem):
        c = lax.axis_index("c")
        s = lax.axis_index("s")
        wid = c * NS + s
        base = wid * EPW

        pltpu.sync_copy(zeros_hbm.at[pl.ds(s * RPS, RPS)],
                        acc_sh.at[pl.ds(s * RPS, RPS)])
        plsc.subcore_barrier()

        pltpu.sync_copy(dst_hbm.at[wid], idx_v)
        pltpu.sync_copy(msg_hbm.at[pl.ds(base, EPW)], rows_v)

        def _scat(j, carry):
            pltpu.sync_copy(rows_v.at[pl.ds(j * GCH, GCH)],
                            acc_sh.at[idx_v.at[j]], add=True)
            return carry

        lax.fori_loop(0, NCH, _scat, 0)
        plsc.subcore_barrier()

        pltpu.sync_copy(acc_sh.at[pl.ds(s * RPS, RPS)],
                        part_hbm.at[c, pl.ds(s * RPS, RPS)])

    return _sc_gather, _sc_scatter


def _msg_body(ef_ref, hs_ref, we1_ref, be1_ref, we2_ref, be2_ref, p_ref,
              msg_ref):
    henc = jnp.maximum(
        jnp.dot(ef_ref[...], we1_ref[...], preferred_element_type=jnp.float32)
        + be1_ref[...], 0.0)
    ewt = jnp.dot(henc, we2_ref[...],
                  preferred_element_type=jnp.float32) + be2_ref[...]
    hrep = jnp.dot(hs_ref[...], p_ref[...], preferred_element_type=jnp.float32)
    prod = hrep * ewt
    prod = prod[:, :512] + prod[:, 512:]
    prod = prod[:, :256] + prod[:, 256:]
    prod = prod[:, :128] + prod[:, 128:]
    prod = prod[:, :64] + prod[:, 64:]
    msg_ref[...] = prod[:, :32] + prod[:, 32:]


def _msg_kernel(e_feat_p, h_src, We1, be1, We2, be2, P):
    grid = (E_PAD // TE,)
    return pl.pallas_call(
        _msg_body,
        grid=grid,
        in_specs=[
            pl.BlockSpec((TE, DE), lambda i: (i, 0)),
            pl.BlockSpec((TE, D), lambda i: (i, 0)),
            pl.BlockSpec((DE, DH), lambda i: (0, 0)),
            pl.BlockSpec((1, DH), lambda i: (0, 0)),
            pl.BlockSpec((DH, D * D), lambda i: (0, 0)),
            pl.BlockSpec((1, D * D), lambda i: (0, 0)),
            pl.BlockSpec((D, D * D), lambda i: (0, 0)),
        ],
        out_specs=pl.BlockSpec((TE, D), lambda i: (i, 0)),
        out_shape=jax.ShapeDtypeStruct((E_PAD, D), jnp.float32),
    )(e_feat_p, h_src, We1, be1, We2, be2, P)


def _prologue_body(nf_ref, w0_ref, b0_ref, out_ref):
    out_ref[...] = jnp.maximum(
        jnp.dot(nf_ref[...], w0_ref[...], preferred_element_type=jnp.float32)
        + b0_ref[...], 0.0)


def _prologue(n_feat, W0, b0):
    return pl.pallas_call(
        _prologue_body,
        out_shape=jax.ShapeDtypeStruct((N, D), jnp.float32),
    )(n_feat, W0, b0)


def _update_body(p0_ref, p1_ref, out_ref, cb_ref, wm1_ref, wm2_ref, bm_ref,
                 o_ref):
    out = out_ref[...]
    m = jnp.maximum(p0_ref[...] + p1_ref[...] + out + cb_ref[...], 0.0)
    o_ref[...] = (jnp.dot(m, wm1_ref[...], preferred_element_type=jnp.float32)
                  + jnp.dot(out, wm2_ref[...],
                            preferred_element_type=jnp.float32)
                  + bm_ref[...])


def _final_body(p0_ref, p1_ref, out_ref, cb_ref, wm1_ref, wm2_ref, bm_ref,
                init_ref, o_ref):
    out = out_ref[...]
    m = jnp.maximum(p0_ref[...] + p1_ref[...] + out + cb_ref[...], 0.0)
    o_ref[...] = (jnp.dot(m, wm1_ref[...], preferred_element_type=jnp.float32)
                  + jnp.dot(out, wm2_ref[...],
                            preferred_element_type=jnp.float32)
                  + bm_ref[...] + init_ref[...])


def _update(p0, p1, out, cb, wm1, wm2, bm):
    return pl.pallas_call(
        _update_body,
        out_shape=jax.ShapeDtypeStruct((N, D), jnp.float32),
    )(p0, p1, out, cb, wm1, wm2, bm)


def _final(p0, p1, out, cb, wm1, wm2, bm, init):
    return pl.pallas_call(
        _final_body,
        out_shape=jax.ShapeDtypeStruct((N, D), jnp.float32),
    )(p0, p1, out, cb, wm1, wm2, bm, init)


def kernel(edge_index, n_feat, e_feat, W0, b0, We1, be1, We2, be2, conv_bias,
           Wm, bm):
    src = edge_index[0]
    dst = edge_index[1]
    npad = E_PAD - E
    fill = (jnp.arange(npad, dtype=jnp.int32) % (N_ACC - N))
    src_p = jnp.concatenate([src, fill]).reshape(NW, NCH, GCH)
    dst_p = jnp.concatenate([dst, N + fill]).reshape(NW, NCH, GCH)
    e_feat_p = jnp.concatenate(
        [e_feat, jnp.zeros((npad, DE), jnp.float32)], axis=0)
    zeros_acc = jnp.zeros((N_ACC, D), jnp.float32)
    be1_2 = be1.reshape(1, DH)
    be2_2 = be2.reshape(1, D * D)
    cb_2 = conv_bias.reshape(1, D)
    bm_2 = bm.reshape(1, D)
    b0_2 = b0.reshape(1, D)
    wm1 = Wm[:D]
    wm2 = Wm[D:]
    P = jnp.kron(jnp.eye(D, dtype=jnp.float32), jnp.ones((1, D), jnp.float32))

    sc_gather, sc_scatter = _sc_kernels()
    out = _prologue(n_feat, W0, b0_2)
    for step in range(STEPS):
        h_src = sc_gather(out, src_p)
        msg = _msg_kernel(e_feat_p, h_src, We1, be1_2, We2, be2_2, P)
        parts = sc_scatter(msg, dst_p, zeros_acc)
        p0 = parts[0, :N]
        p1 = parts[1, :N]
        if step == STEPS - 1:
            out = _final(p0, p1, out, cb_2, wm1, wm2, bm_2, n_feat)
        else:
            out = _update(p0, p1, out, cb_2, wm1, wm2, bm_2)
    return out
